# baseline (speedup 1.0000x reference)
import functools

import jax
import jax.numpy as jnp
from jax import lax
from jax.experimental import pallas as pl
from jax.experimental.pallas import tpu as pltpu
from jax.experimental.pallas import tpu_sc as plsc

_N = 10000
_E = 320000
_D = 128
_P1 = 144
_CH = 100
_NC = 2
_NS = 16
_NW = _NC * _NS
_CPW = _E // _CH // _NW
_RPT = _N // _NS
_BN = 1000


def _make_segsum(with_deg):
  mesh = plsc.VectorSubcoreMesh(core_axis_name="c", subcore_axis_name="s")
  out_type = [jax.ShapeDtypeStruct((_NC, _N, _D), jnp.float32)]
  scratch = [
      pltpu.VMEM_SHARED((_N, _D), jnp.float32),
      pltpu.VMEM((_CH, _D), jnp.float32),
      pltpu.VMEM((_CH, _D), jnp.float32),
      pltpu.VMEM((_CPW // 2, _CH), jnp.int32),
      pltpu.VMEM((_CPW // 2, _CH), jnp.int32),
      pltpu.SemaphoreType.DMA,
      pltpu.SemaphoreType.DMA,
  ]
  if with_deg:
    out_type.append(jax.ShapeDtypeStruct((_NC, _N, 16), jnp.float32))
    scratch += [
        pltpu.VMEM_SHARED((_N, 16), jnp.float32),
        pltpu.VMEM((_CH, 16), jnp.float32),
        pltpu.SemaphoreType.DMA,
    ]

  @functools.partial(
      pl.kernel,
      out_type=out_type,
      mesh=mesh,
      compiler_params=pltpu.CompilerParams(use_tc_tiling_on_sc=False),
      scratch_types=scratch,
  )
  def seg(*refs):
    if with_deg:
      (x_hbm, edge_hbm, out_hbm, out16_hbm, acc, rows0, rows1,
       idxsrc, idxdst, sem0, sem1, acc16, obuf, sem2) = refs
    else:
      (x_hbm, edge_hbm, out_hbm, acc, rows0, rows1,
       idxsrc, idxdst, sem0, sem1) = refs
    c = lax.axis_index("c")
    s = lax.axis_index("s")
    w = s * _NC + c
    rbuf = (rows0, rows1)
    sems = (sem0, sem1)

    z = jnp.zeros((16,), jnp.float32)

    def zero_row(i, carry):
      for j in range(_D // 16):
        rows0[i, pl.ds(j * 16, 16)] = z
      return carry

    lax.fori_loop(0, _CH, zero_row, 0)

    base = s * _RPT
    nfull = _RPT // _CH
    rem = _RPT % _CH
    for k in range(nfull):
      pltpu.sync_copy(rows0, acc.at[pl.ds(base + k * _CH, _CH)])
    if rem:
      pltpu.sync_copy(rows0.at[pl.ds(0, rem)],
                      acc.at[pl.ds(base + nfull * _CH, rem)])
    if with_deg:
      def zero_o(i, carry):
        obuf[i, pl.ds(0, 16)] = z
        return carry

      lax.fori_loop(0, _CH, zero_o, 0)
      for k in range(nfull):
        pltpu.sync_copy(obuf, acc16.at[pl.ds(base + k * _CH, _CH)])
      if rem:
        pltpu.sync_copy(obuf.at[pl.ds(0, rem)],
                        acc16.at[pl.ds(base + nfull * _CH, rem)])
      one = jnp.ones((16,), jnp.float32)

      def ones_o(i, carry):
        obuf[i, pl.ds(0, 16)] = one
        return carry

      lax.fori_loop(0, _CH, ones_o, 0)
    plsc.subcore_barrier()

    def gstart(j, b):
      pltpu.async_copy(x_hbm.at[idxsrc.at[j]], rbuf[b], sems[b])

    def gwait(j, b):
      pltpu.make_async_copy(x_hbm.at[idxsrc.at[j]], rbuf[b], sems[b]).wait()

    def scat(j, b):
      pltpu.sync_copy(rbuf[b], acc.at[idxdst.at[j]], add=True)

    ih = _CPW // 2
    for half in range(2):
      pltpu.sync_copy(edge_hbm.at[0, pl.ds(w * _CPW + half * ih, ih)], idxsrc)
      pltpu.sync_copy(edge_hbm.at[1, pl.ds(w * _CPW + half * ih, ih)], idxdst)
      gstart(0, 0)
      gstart(1, 1)

      def dfire(j):
        if with_deg:
          pltpu.async_copy(obuf, acc16.at[idxdst.at[j]], sem2, add=True)

      def body(t, carry):
        j = t * 2
        gwait(j, 0)
        scat(j, 0)
        dfire(j)
        gstart(j + 2, 0)
        gwait(j + 1, 1)
        scat(j + 1, 1)
        dfire(j + 1)
        gstart(j + 3, 1)
        return carry

      lax.fori_loop(0, ih // 2 - 1, body, 0)
      gwait(ih - 2, 0)
      scat(ih - 2, 0)
      dfire(ih - 2)
      gwait(ih - 1, 1)
      scat(ih - 1, 1)
      dfire(ih - 1)

      if with_deg:
        def ddrain(j, carry):
          pltpu.make_async_copy(obuf, acc16.at[idxdst.at[0]], sem2).wait()
          return carry

        lax.fori_loop(0, ih, ddrain, 0)
    plsc.subcore_barrier()

    for k in range(nfull):
      pltpu.sync_copy(acc.at[pl.ds(base + k * _CH, _CH)], rows0)
      pltpu.sync_copy(rows0, out_hbm.at[c, pl.ds(base + k * _CH, _CH)])
    if rem:
      off = base + nfull * _CH
      pltpu.sync_copy(acc.at[pl.ds(off, rem)], rows0.at[pl.ds(0, rem)])
      pltpu.sync_copy(rows0.at[pl.ds(0, rem)], out_hbm.at[c, pl.ds(off, rem)])
    if with_deg:
      for k in range(nfull):
        pltpu.sync_copy(acc16.at[pl.ds(base + k * _CH, _CH)], obuf)
        pltpu.sync_copy(obuf, out16_hbm.at[c, pl.ds(base + k * _CH, _CH)])
      if rem:
        off = base + nfull * _CH
        pltpu.sync_copy(acc16.at[pl.ds(off, rem)], obuf.at[pl.ds(0, rem)])
        pltpu.sync_copy(obuf.at[pl.ds(0, rem)],
                        out16_hbm.at[c, pl.ds(off, rem)])

  return seg


_seg1 = _make_segsum(True)
_seg2 = _make_segsum(False)


def _mm_a(h, W1, b1):
  def body(h_ref, w_ref, b_ref, o_ref):
    o_ref[...] = jnp.dot(h_ref[...], w_ref[...],
                         preferred_element_type=jnp.float32) + b_ref[...]

  return pl.pallas_call(
      body,
      grid=(_N // _BN,),
      in_specs=[
          pl.BlockSpec((_BN, _D), lambda i: (i, 0)),
          pl.BlockSpec((_D, _D), lambda i: (0, 0)),
          pl.BlockSpec((1, _D), lambda i: (0, 0)),
      ],
      out_specs=pl.BlockSpec((_BN, _D), lambda i: (i, 0)),
      out_shape=jax.ShapeDtypeStruct((_N, _D), jnp.float32),
  )(h, W1, b1)


def _mm_b(acc1, deg16, W2, b2):
  def body(a_ref, d_ref, w_ref, b_ref, x2_ref):
    a = a_ref[...]
    d = d_ref[...]
    deg = jnp.maximum(d[0, :, 0:1] + d[1, :, 0:1], 1.0)
    h1 = jnp.maximum((a[0] + a[1]) / deg, 0.0)
    x2_ref[...] = jnp.dot(h1, w_ref[...],
                          preferred_element_type=jnp.float32) + b_ref[...]

  return pl.pallas_call(
      body,
      grid=(_N // _BN,),
      in_specs=[
          pl.BlockSpec((_NC, _BN, _D), lambda i: (0, i, 0)),
          pl.BlockSpec((_NC, _BN, 16), lambda i: (0, i, 0)),
          pl.BlockSpec((_D, _D), lambda i: (0, 0)),
          pl.BlockSpec((1, _D), lambda i: (0, 0)),
      ],
      out_specs=pl.BlockSpec((_BN, _D), lambda i: (i, 0)),
      out_shape=jax.ShapeDtypeStruct((_N, _D), jnp.float32),
  )(acc1, deg16, W2, b2)


def _mm_c(acc2, deg16):
  def body(a_ref, d_ref, o_ref):
    a = a_ref[...]
    d = d_ref[...]
    deg = jnp.maximum(d[0, :, 0:1] + d[1, :, 0:1], 1.0)
    h2 = jnp.maximum((a[0] + a[1]) / deg, 0.0)
    nrm = jnp.sqrt(jnp.sum(h2 * h2, axis=1, keepdims=True))
    o_ref[...] = h2 / jnp.maximum(nrm, 1e-12)

  return pl.pallas_call(
      body,
      grid=(_N // _BN,),
      in_specs=[
          pl.BlockSpec((_NC, _BN, _D), lambda i: (0, i, 0)),
          pl.BlockSpec((_NC, _BN, 16), lambda i: (0, i, 0)),
      ],
      out_specs=pl.BlockSpec((_BN, _D), lambda i: (i, 0)),
      out_shape=jax.ShapeDtypeStruct((_N, _D), jnp.float32),
  )(acc2, deg16)


def kernel(h, edge_index, W1, b1, W2, b2):
  edge3 = edge_index.reshape(2, _E // _CH, _CH)
  x1 = _mm_a(h, W1, b1.reshape(1, _D))
  acc1, deg16 = _seg1(x1, edge3)
  x2 = _mm_b(acc1, deg16, W2, b2.reshape(1, _D))
  acc2, = _seg2(x2, edge3)
  return _mm_c(acc2, deg16)


# speedup vs baseline: 1.0120x; 1.0120x over previous
import functools

import jax
import jax.numpy as jnp
from jax import lax
from jax.experimental import pallas as pl
from jax.experimental.pallas import tpu as pltpu
from jax.experimental.pallas import tpu_sc as plsc

_N = 10000
_E = 320000
_D = 128
_CH = 128
_NC = 2
_NS = 16
_NW = _NC * _NS
_NR = _E // _CH
_CPW = _NR // _NW
_XTRA = _NR - _CPW * _NW
_IH0 = 20
_DW = 16
_RPT = _N // _NS
_BN = 1000


def _make_segsum(with_deg):
  mesh = plsc.VectorSubcoreMesh(core_axis_name="c", subcore_axis_name="s")
  out_type = [jax.ShapeDtypeStruct((_NC, _N, _D), jnp.float32)]
  scratch = [
      pltpu.VMEM_SHARED((_N, _D), jnp.float32),
      pltpu.VMEM((_CH, _D), jnp.float32),
      pltpu.VMEM((_CH, _D), jnp.float32),
      pltpu.VMEM((_IH0, _CH), jnp.int32),
      pltpu.VMEM((_IH0, _CH), jnp.int32),
      pltpu.SemaphoreType.DMA,
      pltpu.SemaphoreType.DMA,
  ]
  if with_deg:
    out_type.append(jax.ShapeDtypeStruct((_NC, _N, _DW), jnp.float32))
    scratch += [
        pltpu.VMEM_SHARED((_N, _DW), jnp.float32),
        pltpu.VMEM((_CH, _DW), jnp.float32),
        pltpu.SemaphoreType.DMA,
    ]

  @functools.partial(
      pl.kernel,
      out_type=out_type,
      mesh=mesh,
      compiler_params=pltpu.CompilerParams(use_tc_tiling_on_sc=False),
      scratch_types=scratch,
  )
  def seg(*refs):
    if with_deg:
      (x_hbm, edge_hbm, out_hbm, out16_hbm, acc, rows0, rows1,
       idxsrc, idxdst, sem0, sem1, acc16, obuf, sem2) = refs
    else:
      (x_hbm, edge_hbm, out_hbm, acc, rows0, rows1,
       idxsrc, idxdst, sem0, sem1) = refs
    c = lax.axis_index("c")
    s = lax.axis_index("s")
    w = s * _NC + c
    rbuf = (rows0, rows1)
    sems = (sem0, sem1)

    z = jnp.zeros((16,), jnp.float32)

    def zero_row(i, carry):
      for j in range(_D // 16):
        rows0[i, pl.ds(j * 16, 16)] = z
      return carry

    lax.fori_loop(0, _CH, zero_row, 0)

    base = s * _RPT
    nfull = _RPT // _CH
    rem = _RPT % _CH
    for k in range(nfull):
      pltpu.sync_copy(rows0, acc.at[pl.ds(base + k * _CH, _CH)])
    if rem:
      pltpu.sync_copy(rows0.at[pl.ds(0, rem)],
                      acc.at[pl.ds(base + nfull * _CH, rem)])
    if with_deg:
      def zero_o(i, carry):
        obuf[i, pl.ds(0, _DW)] = z
        return carry

      lax.fori_loop(0, _CH, zero_o, 0)
      for k in range(nfull):
        pltpu.sync_copy(obuf, acc16.at[pl.ds(base + k * _CH, _CH)])
      if rem:
        pltpu.sync_copy(obuf.at[pl.ds(0, rem)],
                        acc16.at[pl.ds(base + nfull * _CH, rem)])
      one = jnp.ones((16,), jnp.float32)

      def ones_o(i, carry):
        obuf[i, pl.ds(0, _DW)] = one
        return carry

      lax.fori_loop(0, _CH, ones_o, 0)
    plsc.subcore_barrier()

    def gstart(j, b):
      pltpu.async_copy(x_hbm.at[idxsrc.at[j]], rbuf[b], sems[b])

    def gwait(j, b):
      pltpu.make_async_copy(x_hbm.at[idxsrc.at[j]], rbuf[b], sems[b]).wait()

    def scat(j, b):
      pltpu.sync_copy(rbuf[b], acc.at[idxdst.at[j]], add=True)

    def dfire(j):
      if with_deg:
        pltpu.async_copy(obuf, acc16.at[idxdst.at[j]], sem2, add=True)

    def ddrain(n):
      if with_deg:
        def dw(j, carry):
          pltpu.make_async_copy(obuf, acc16.at[idxdst.at[0]], sem2).wait()
          return carry

        lax.fori_loop(0, n, dw, 0)

    stages = []
    roff = 0
    while roff < _CPW:
      stages.append((roff, min(_IH0, _CPW - roff)))
      roff += _IH0
    for roff, ih in stages:
      pltpu.sync_copy(edge_hbm.at[0, pl.ds(w * _CPW + roff, ih)],
                      idxsrc.at[pl.ds(0, ih)])
      pltpu.sync_copy(edge_hbm.at[1, pl.ds(w * _CPW + roff, ih)],
                      idxdst.at[pl.ds(0, ih)])
      gstart(0, 0)
      gstart(1, 1)

      def body(t, carry):
        j = t * 2
        gwait(j, 0)
        scat(j, 0)
        dfire(j)
        gstart(j + 2, 0)
        gwait(j + 1, 1)
        scat(j + 1, 1)
        dfire(j + 1)
        gstart(j + 3, 1)
        return carry

      lax.fori_loop(0, ih // 2 - 1, body, 0)
      gwait(ih - 2, 0)
      scat(ih - 2, 0)
      dfire(ih - 2)
      gwait(ih - 1, 1)
      scat(ih - 1, 1)
      dfire(ih - 1)
      ddrain(ih)

    @pl.when(w < _XTRA)
    def _extra():
      pltpu.sync_copy(edge_hbm.at[0, pl.ds(_CPW * _NW + w, 1)],
                      idxsrc.at[pl.ds(0, 1)])
      pltpu.sync_copy(edge_hbm.at[1, pl.ds(_CPW * _NW + w, 1)],
                      idxdst.at[pl.ds(0, 1)])
      gstart(0, 0)
      gwait(0, 0)
      scat(0, 0)
      dfire(0)
      ddrain(1)

    plsc.subcore_barrier()

    for k in range(nfull):
      pltpu.sync_copy(acc.at[pl.ds(base + k * _CH, _CH)], rows0)
      pltpu.sync_copy(rows0, out_hbm.at[c, pl.ds(base + k * _CH, _CH)])
    if rem:
      off = base + nfull * _CH
      pltpu.sync_copy(acc.at[pl.ds(off, rem)], rows0.at[pl.ds(0, rem)])
      pltpu.sync_copy(rows0.at[pl.ds(0, rem)], out_hbm.at[c, pl.ds(off, rem)])
    if with_deg:
      for k in range(nfull):
        pltpu.sync_copy(acc16.at[pl.ds(base + k * _CH, _CH)], obuf)
        pltpu.sync_copy(obuf, out16_hbm.at[c, pl.ds(base + k * _CH, _CH)])
      if rem:
        off = base + nfull * _CH
        pltpu.sync_copy(acc16.at[pl.ds(off, rem)], obuf.at[pl.ds(0, rem)])
        pltpu.sync_copy(obuf.at[pl.ds(0, rem)],
                        out16_hbm.at[c, pl.ds(off, rem)])

  return seg


_seg1 = _make_segsum(True)
_seg2 = _make_segsum(False)


def _mm_a(h, W1, b1):
  def body(h_ref, w_ref, b_ref, o_ref):
    o_ref[...] = jnp.dot(h_ref[...], w_ref[...],
                         preferred_element_type=jnp.float32) + b_ref[...]

  return pl.pallas_call(
      body,
      grid=(_N // _BN,),
      in_specs=[
          pl.BlockSpec((_BN, _D), lambda i: (i, 0)),
          pl.BlockSpec((_D, _D), lambda i: (0, 0)),
          pl.BlockSpec((1, _D), lambda i: (0, 0)),
      ],
      out_specs=pl.BlockSpec((_BN, _D), lambda i: (i, 0)),
      out_shape=jax.ShapeDtypeStruct((_N, _D), jnp.float32),
  )(h, W1, b1)


def _mm_b(acc1, deg16, W2, b2):
  def body(a_ref, d_ref, w_ref, b_ref, x2_ref):
    a = a_ref[...]
    d = d_ref[...]
    deg = jnp.maximum(d[0, :, 0:1] + d[1, :, 0:1], 1.0)
    h1 = jnp.maximum((a[0] + a[1]) / deg, 0.0)
    x2_ref[...] = jnp.dot(h1, w_ref[...],
                          preferred_element_type=jnp.float32) + b_ref[...]

  return pl.pallas_call(
      body,
      grid=(_N // _BN,),
      in_specs=[
          pl.BlockSpec((_NC, _BN, _D), lambda i: (0, i, 0)),
          pl.BlockSpec((_NC, _BN, 16), lambda i: (0, i, 0)),
          pl.BlockSpec((_D, _D), lambda i: (0, 0)),
          pl.BlockSpec((1, _D), lambda i: (0, 0)),
      ],
      out_specs=pl.BlockSpec((_BN, _D), lambda i: (i, 0)),
      out_shape=jax.ShapeDtypeStruct((_N, _D), jnp.float32),
  )(acc1, deg16, W2, b2)


def _mm_c(acc2, deg16):
  def body(a_ref, d_ref, o_ref):
    a = a_ref[...]
    d = d_ref[...]
    deg = jnp.maximum(d[0, :, 0:1] + d[1, :, 0:1], 1.0)
    h2 = jnp.maximum((a[0] + a[1]) / deg, 0.0)
    nrm = jnp.sqrt(jnp.sum(h2 * h2, axis=1, keepdims=True))
    o_ref[...] = h2 / jnp.maximum(nrm, 1e-12)

  return pl.pallas_call(
      body,
      grid=(_N // _BN,),
      in_specs=[
          pl.BlockSpec((_NC, _BN, _D), lambda i: (0, i, 0)),
          pl.BlockSpec((_NC, _BN, 16), lambda i: (0, i, 0)),
      ],
      out_specs=pl.BlockSpec((_BN, _D), lambda i: (i, 0)),
      out_shape=jax.ShapeDtypeStruct((_N, _D), jnp.float32),
  )(acc2, deg16)


def kernel(h, edge_index, W1, b1, W2, b2):
  edge3 = edge_index.reshape(2, _E // _CH, _CH)
  x1 = _mm_a(h, W1, b1.reshape(1, _D))
  acc1, deg16 = _seg1(x1, edge3)
  x2 = _mm_b(acc1, deg16, W2, b2.reshape(1, _D))
  acc2, = _seg2(x2, edge3)
  return _mm_c(acc2, deg16)


# speedup vs baseline: 1.0353x; 1.0230x over previous
import functools

import jax
import jax.numpy as jnp
from jax import lax
from jax.experimental import pallas as pl
from jax.experimental.pallas import tpu as pltpu
from jax.experimental.pallas import tpu_sc as plsc

_N = 10000
_E = 320000
_D = 128
_CH = 128
_NC = 2
_NS = 16
_NW = _NC * _NS
_NR = _E // _CH
_CPW = _NR // _NW
_XTRA = _NR - _CPW * _NW
_IH0 = 20
_DW = 16
_RPT = _N // _NS
_BN = 2000


def _make_segsum(with_deg):
  mesh = plsc.VectorSubcoreMesh(core_axis_name="c", subcore_axis_name="s")
  out_type = [jax.ShapeDtypeStruct((_NC, _N, _D), jnp.float32)]
  scratch = [
      pltpu.VMEM_SHARED((_N, _D), jnp.float32),
      pltpu.VMEM((_CH, _D), jnp.float32),
      pltpu.VMEM((_CH, _D), jnp.float32),
      pltpu.VMEM((_IH0, _CH), jnp.int32),
      pltpu.VMEM((_IH0, _CH), jnp.int32),
      pltpu.SemaphoreType.DMA,
      pltpu.SemaphoreType.DMA,
  ]
  if with_deg:
    out_type.append(jax.ShapeDtypeStruct((_NC, _N, _DW), jnp.float32))
    scratch += [
        pltpu.VMEM_SHARED((_N, _DW), jnp.float32),
        pltpu.VMEM((_CH, _DW), jnp.float32),
        pltpu.SemaphoreType.DMA,
    ]

  @functools.partial(
      pl.kernel,
      out_type=out_type,
      mesh=mesh,
      compiler_params=pltpu.CompilerParams(use_tc_tiling_on_sc=False),
      scratch_types=scratch,
  )
  def seg(*refs):
    if with_deg:
      (x_hbm, edge_hbm, out_hbm, out16_hbm, acc, rows0, rows1,
       idxsrc, idxdst, sem0, sem1, acc16, obuf, sem2) = refs
    else:
      (x_hbm, edge_hbm, out_hbm, acc, rows0, rows1,
       idxsrc, idxdst, sem0, sem1) = refs
    c = lax.axis_index("c")
    s = lax.axis_index("s")
    w = s * _NC + c
    rbuf = (rows0, rows1)
    sems = (sem0, sem1)

    z = jnp.zeros((16,), jnp.float32)

    def zero_row(i, carry):
      for j in range(_D // 16):
        rows0[i, pl.ds(j * 16, 16)] = z
      return carry

    lax.fori_loop(0, _CH, zero_row, 0)

    base = s * _RPT
    nfull = _RPT // _CH
    rem = _RPT % _CH
    for k in range(nfull):
      pltpu.sync_copy(rows0, acc.at[pl.ds(base + k * _CH, _CH)])
    if rem:
      pltpu.sync_copy(rows0.at[pl.ds(0, rem)],
                      acc.at[pl.ds(base + nfull * _CH, rem)])
    if with_deg:
      def zero_o(i, carry):
        obuf[i, pl.ds(0, _DW)] = z
        return carry

      lax.fori_loop(0, _CH, zero_o, 0)
      for k in range(nfull):
        pltpu.sync_copy(obuf, acc16.at[pl.ds(base + k * _CH, _CH)])
      if rem:
        pltpu.sync_copy(obuf.at[pl.ds(0, rem)],
                        acc16.at[pl.ds(base + nfull * _CH, rem)])
      one = jnp.ones((16,), jnp.float32)

      def ones_o(i, carry):
        obuf[i, pl.ds(0, _DW)] = one
        return carry

      lax.fori_loop(0, _CH, ones_o, 0)
    plsc.subcore_barrier()

    def gstart(j, b):
      pltpu.async_copy(x_hbm.at[idxsrc.at[j]], rbuf[b], sems[b])

    def gwait(j, b):
      pltpu.make_async_copy(x_hbm.at[idxsrc.at[j]], rbuf[b], sems[b]).wait()

    def scat(j, b):
      pltpu.sync_copy(rbuf[b], acc.at[idxdst.at[j]], add=True)

    def dfire(j):
      if with_deg:
        pltpu.async_copy(obuf, acc16.at[idxdst.at[j]], sem2, add=True)

    def ddrain(n):
      if with_deg:
        def dw(j, carry):
          pltpu.make_async_copy(obuf, acc16.at[idxdst.at[0]], sem2).wait()
          return carry

        lax.fori_loop(0, n, dw, 0)

    stages = []
    roff = 0
    while roff < _CPW:
      stages.append((roff, min(_IH0, _CPW - roff)))
      roff += _IH0
    for roff, ih in stages:
      pltpu.sync_copy(edge_hbm.at[0, pl.ds(w * _CPW + roff, ih)],
                      idxsrc.at[pl.ds(0, ih)])
      pltpu.sync_copy(edge_hbm.at[1, pl.ds(w * _CPW + roff, ih)],
                      idxdst.at[pl.ds(0, ih)])
      gstart(0, 0)
      gstart(1, 1)

      def body(t, carry):
        j = t * 2
        gwait(j, 0)
        scat(j, 0)
        dfire(j)
        gstart(j + 2, 0)
        gwait(j + 1, 1)
        scat(j + 1, 1)
        dfire(j + 1)
        gstart(j + 3, 1)
        return carry

      lax.fori_loop(0, ih // 2 - 1, body, 0)
      gwait(ih - 2, 0)
      scat(ih - 2, 0)
      dfire(ih - 2)
      gwait(ih - 1, 1)
      scat(ih - 1, 1)
      dfire(ih - 1)
      ddrain(ih)

    @pl.when(w < _XTRA)
    def _extra():
      pltpu.sync_copy(edge_hbm.at[0, pl.ds(_CPW * _NW + w, 1)],
                      idxsrc.at[pl.ds(0, 1)])
      pltpu.sync_copy(edge_hbm.at[1, pl.ds(_CPW * _NW + w, 1)],
                      idxdst.at[pl.ds(0, 1)])
      gstart(0, 0)
      gwait(0, 0)
      scat(0, 0)
      dfire(0)
      ddrain(1)

    plsc.subcore_barrier()

    for k in range(nfull):
      pltpu.sync_copy(acc.at[pl.ds(base + k * _CH, _CH)], rows0)
      pltpu.sync_copy(rows0, out_hbm.at[c, pl.ds(base + k * _CH, _CH)])
    if rem:
      off = base + nfull * _CH
      pltpu.sync_copy(acc.at[pl.ds(off, rem)], rows0.at[pl.ds(0, rem)])
      pltpu.sync_copy(rows0.at[pl.ds(0, rem)], out_hbm.at[c, pl.ds(off, rem)])
    if with_deg:
      for k in range(nfull):
        pltpu.sync_copy(acc16.at[pl.ds(base + k * _CH, _CH)], obuf)
        pltpu.sync_copy(obuf, out16_hbm.at[c, pl.ds(base + k * _CH, _CH)])
      if rem:
        off = base + nfull * _CH
        pltpu.sync_copy(acc16.at[pl.ds(off, rem)], obuf.at[pl.ds(0, rem)])
        pltpu.sync_copy(obuf.at[pl.ds(0, rem)],
                        out16_hbm.at[c, pl.ds(off, rem)])

  return seg


_seg1 = _make_segsum(True)
_seg2 = _make_segsum(False)


def _mm_a(h, W1, b1):
  def body(h_ref, w_ref, b_ref, o_ref):
    o_ref[...] = jnp.dot(h_ref[...], w_ref[...],
                         preferred_element_type=jnp.float32) + b_ref[...]

  return pl.pallas_call(
      body,
      grid=(_N // _BN,),
      in_specs=[
          pl.BlockSpec((_BN, _D), lambda i: (i, 0)),
          pl.BlockSpec((_D, _D), lambda i: (0, 0)),
          pl.BlockSpec((1, _D), lambda i: (0, 0)),
      ],
      out_specs=pl.BlockSpec((_BN, _D), lambda i: (i, 0)),
      out_shape=jax.ShapeDtypeStruct((_N, _D), jnp.float32),
  )(h, W1, b1)


def _mm_b(acc1, deg16, W2, b2):
  def body(a_ref, d_ref, w_ref, b_ref, x2_ref):
    a = a_ref[...]
    d = d_ref[...]
    deg = jnp.maximum(d[0, :, 0:1] + d[1, :, 0:1], 1.0)
    h1 = jnp.maximum((a[0] + a[1]) / deg, 0.0)
    x2_ref[...] = jnp.dot(h1, w_ref[...],
                          preferred_element_type=jnp.float32) + b_ref[...]

  return pl.pallas_call(
      body,
      grid=(_N // _BN,),
      in_specs=[
          pl.BlockSpec((_NC, _BN, _D), lambda i: (0, i, 0)),
          pl.BlockSpec((_NC, _BN, 16), lambda i: (0, i, 0)),
          pl.BlockSpec((_D, _D), lambda i: (0, 0)),
          pl.BlockSpec((1, _D), lambda i: (0, 0)),
      ],
      out_specs=pl.BlockSpec((_BN, _D), lambda i: (i, 0)),
      out_shape=jax.ShapeDtypeStruct((_N, _D), jnp.float32),
  )(acc1, deg16, W2, b2)


def _mm_c(acc2, deg16):
  def body(a_ref, d_ref, o_ref):
    a = a_ref[...]
    d = d_ref[...]
    deg = jnp.maximum(d[0, :, 0:1] + d[1, :, 0:1], 1.0)
    h2 = jnp.maximum((a[0] + a[1]) / deg, 0.0)
    nrm = jnp.sqrt(jnp.sum(h2 * h2, axis=1, keepdims=True))
    o_ref[...] = h2 / jnp.maximum(nrm, 1e-12)

  return pl.pallas_call(
      body,
      grid=(_N // _BN,),
      in_specs=[
          pl.BlockSpec((_NC, _BN, _D), lambda i: (0, i, 0)),
          pl.BlockSpec((_NC, _BN, 16), lambda i: (0, i, 0)),
      ],
      out_specs=pl.BlockSpec((_BN, _D), lambda i: (i, 0)),
      out_shape=jax.ShapeDtypeStruct((_N, _D), jnp.float32),
  )(acc2, deg16)


def kernel(h, edge_index, W1, b1, W2, b2):
  edge3 = edge_index.reshape(2, _E // _CH, _CH)
  x1 = _mm_a(h, W1, b1.reshape(1, _D))
  acc1, deg16 = _seg1(x1, edge3)
  x2 = _mm_b(acc1, deg16, W2, b2.reshape(1, _D))
  acc2, = _seg2(x2, edge3)
  return _mm_c(acc2, deg16)
